# Initial kernel scaffold; baseline (speedup 1.0000x reference)
import jax
import jax.numpy as jnp
from jax.experimental import pallas as pl


def kernel(h, edge_index, params):
    raise NotImplementedError("write your pallas kernel here")


# speedup vs baseline: 4.2639x; 4.2639x over previous
import functools

import jax
import jax.numpy as jnp
from jax import lax
from jax.experimental import pallas as pl
from jax.experimental.pallas import tpu as pltpu
from jax.experimental.pallas import tpu_sc as plsc

N = 10000
D = 128
E = 320000
NUM_LAYERS = 4

NCORES = 2
NSUB = 16
NW = NCORES * NSUB
ROWS_PER_TILE = 632
NPAD = ROWS_PER_TILE * NSUB
EPW = E // NW
CHUNK = 80
NCHUNKS = EPW // CHUNK


def _aggregate_sc(hh, src, dst):
    mesh = plsc.VectorSubcoreMesh(core_axis_name="c", subcore_axis_name="s")

    @functools.partial(
        pl.kernel,
        mesh=mesh,
        out_type=jax.ShapeDtypeStruct((NCORES * NPAD, D), jnp.float32),
        scratch_types=[
            pltpu.VMEM((CHUNK,), jnp.int32),
            pltpu.VMEM((CHUNK,), jnp.int32),
            pltpu.VMEM((CHUNK, D), jnp.float32),
            pltpu.VMEM((8, D), jnp.float32),
            pltpu.VMEM_SHARED((NPAD, D), jnp.float32),
            pltpu.SemaphoreType.DMA,
        ],
    )
    def k(hh_hbm, src_hbm, dst_hbm, out_hbm, sidx, didx, rows, zbuf, aggsh, sem):
        c = lax.axis_index("c")
        s = lax.axis_index("s")
        wid = c * NSUB + s
        zv = jnp.zeros((16,), jnp.float32)
        for r in range(8):
            for q in range(D // 16):
                zbuf[r, pl.ds(q * 16, 16)] = zv
        row0 = s * ROWS_PER_TILE

        def zbody(j, carry):
            pltpu.sync_copy(zbuf, aggsh.at[pl.ds(row0 + j * 8, 8)])
            return carry

        lax.fori_loop(0, ROWS_PER_TILE // 8, zbody, 0)
        plsc.subcore_barrier()

        ebase = wid * EPW

        def ebody(g, carry):
            off = ebase + g * CHUNK
            pltpu.sync_copy(src_hbm.at[pl.ds(off, CHUNK)], sidx)
            pltpu.async_copy(hh_hbm.at[sidx], rows, sem).wait()
            pltpu.sync_copy(dst_hbm.at[pl.ds(off, CHUNK)], didx)
            pltpu.sync_copy(rows, aggsh.at[didx], add=True)
            return carry

        lax.fori_loop(0, NCHUNKS, ebody, 0)
        plsc.subcore_barrier()
        pltpu.sync_copy(
            aggsh.at[pl.ds(row0, ROWS_PER_TILE)],
            out_hbm.at[pl.ds(c * NPAD + row0, ROWS_PER_TILE)],
        )

    return k(hh, src, dst)


def _mlp_body(hh_ref, agg_ref, w1_ref, g1_ref, b1_ref, w2_ref, go_ref, bo_ref,
              out_ref, pooled_ref, *p0_ref):
    agg = agg_ref[0:N, :] + agg_ref[NPAD:NPAD + N, :]
    z = hh_ref[...] + agg
    z = jnp.dot(z, w1_ref[...], preferred_element_type=jnp.float32,
                precision=lax.Precision.HIGHEST)
    mu = jnp.mean(z, axis=0, keepdims=True)
    var = jnp.mean((z - mu) ** 2, axis=0, keepdims=True)
    z = g1_ref[...] * (z - mu) * lax.rsqrt(var + 1e-5) + b1_ref[...]
    z = jnp.maximum(z, 0.0)
    z = jnp.dot(z, w2_ref[...], preferred_element_type=jnp.float32,
                precision=lax.Precision.HIGHEST)
    mu = jnp.mean(z, axis=0, keepdims=True)
    var = jnp.mean((z - mu) ** 2, axis=0, keepdims=True)
    z = go_ref[...] * (z - mu) * lax.rsqrt(var + 1e-5) + bo_ref[...]
    z = jnp.maximum(z, 0.0)
    out_ref[...] = z
    pooled_ref[...] = jnp.sum(z, axis=0, keepdims=True)
    if p0_ref:
        p0_ref[0][...] = jnp.sum(hh_ref[...], axis=0, keepdims=True)


def _mlp_tc(hh, aggflat, w1, g1, b1, w2, go, bo, first):
    out_shape = [
        jax.ShapeDtypeStruct((N, D), jnp.float32),
        jax.ShapeDtypeStruct((1, D), jnp.float32),
    ]
    if first:
        out_shape.append(jax.ShapeDtypeStruct((1, D), jnp.float32))
    return pl.pallas_call(
        _mlp_body,
        out_shape=out_shape,
    )(hh, aggflat, w1, g1.reshape(1, D), b1.reshape(1, D),
      w2, go.reshape(1, D), bo.reshape(1, D))


def _readout_body(p_ref, wp_ref, bp_ref, out_ref):
    p = p_ref[...]
    wp = wp_ref[...]
    acc = jnp.sum(bp_ref[...], axis=0, keepdims=True)
    for i in range(NUM_LAYERS + 1):
        acc = acc + jnp.dot(p[i:i + 1, :], wp[i], preferred_element_type=jnp.float32,
                            precision=lax.Precision.HIGHEST)
    out_ref[...] = acc


def _readout_tc(pooled_stack, wp_stack, bp_stack):
    return pl.pallas_call(
        _readout_body,
        out_shape=jax.ShapeDtypeStruct((1, D), jnp.float32),
    )(pooled_stack, wp_stack, bp_stack)


def kernel(h, edge_index, params):
    src = edge_index[0]
    dst = edge_index[1]
    hh = h
    pooled = []
    for i in range(NUM_LAYERS):
        aggflat = _aggregate_sc(hh, src, dst)
        outs = _mlp_tc(hh, aggflat, params[f"W1_{i}"], params[f"g1_{i}"],
                       params[f"b1_{i}"], params[f"W2_{i}"], params[f"go_{i}"],
                       params[f"bo_{i}"], first=(i == 0))
        if i == 0:
            hh, p, p0 = outs
            pooled.append(p0)
        else:
            hh, p = outs
        pooled.append(p)
    pooled_stack = jnp.concatenate(pooled, axis=0)
    wp_stack = jnp.stack([params[f"Wp_{i}"] for i in range(NUM_LAYERS + 1)])
    bp_stack = jnp.stack([params[f"bp_{i}"] for i in range(NUM_LAYERS + 1)])
    return _readout_tc(pooled_stack, wp_stack, bp_stack)


# speedup vs baseline: 9.3781x; 2.1994x over previous
import functools

import jax
import jax.numpy as jnp
from jax import lax
from jax.experimental import pallas as pl
from jax.experimental.pallas import tpu as pltpu
from jax.experimental.pallas import tpu_sc as plsc

N = 10000
D = 128
E = 320000
NUM_LAYERS = 4

NCORES = 2
NSUB = 16
NW = NCORES * NSUB
ROWS_PER_TILE = 632
NPAD = ROWS_PER_TILE * NSUB
CHUNK = 128
BLK = 16
NBLK = 5
NCHUNKS = BLK * NBLK
EPW = NCHUNKS * CHUNK
EPAD = NW * EPW


def _aggregate_sc(hh, src2d, dst2d):
    mesh = plsc.VectorSubcoreMesh(core_axis_name="c", subcore_axis_name="s")

    @functools.partial(
        pl.kernel,
        mesh=mesh,
        out_type=jax.ShapeDtypeStruct((NCORES * NPAD, D), jnp.float32),
        scratch_types=[
            pltpu.VMEM((BLK, CHUNK), jnp.int32),
            pltpu.VMEM((BLK, CHUNK), jnp.int32),
            pltpu.VMEM((CHUNK, D), jnp.float32),
            pltpu.VMEM((CHUNK, D), jnp.float32),
            pltpu.VMEM((8, D), jnp.float32),
            pltpu.VMEM_SHARED((NPAD, D), jnp.float32),
            pltpu.SemaphoreType.DMA,
            pltpu.SemaphoreType.DMA,
        ],
    )
    def k(hh_hbm, src_hbm, dst_hbm, out_hbm, sidx, didx, rows_a, rows_b,
          zbuf, aggsh, sem_a, sem_b):
        c = lax.axis_index("c")
        s = lax.axis_index("s")
        wid = c * NSUB + s
        zv = jnp.zeros((16,), jnp.float32)
        for r in range(8):
            for q in range(D // 16):
                zbuf[r, pl.ds(q * 16, 16)] = zv
        row0 = s * ROWS_PER_TILE

        def zbody(j, carry):
            pltpu.sync_copy(zbuf, aggsh.at[pl.ds(row0 + j * 8, 8)])
            return carry

        lax.fori_loop(0, ROWS_PER_TILE // 8, zbody, 0)
        plsc.subcore_barrier()

        def wait_a():
            pltpu.make_async_copy(hh_hbm.at[sidx.at[0]], rows_a, sem_a).wait()

        def wait_b():
            pltpu.make_async_copy(hh_hbm.at[sidx.at[0]], rows_b, sem_b).wait()

        def blk_body(blk, carry):
            pltpu.sync_copy(src_hbm.at[pl.ds(wid * NCHUNKS + blk * BLK, BLK)],
                            sidx)
            pltpu.sync_copy(dst_hbm.at[pl.ds(wid * NCHUNKS + blk * BLK, BLK)],
                            didx)
            pltpu.async_copy(hh_hbm.at[sidx.at[0]], rows_a, sem_a)
            pltpu.async_copy(hh_hbm.at[sidx.at[1]], rows_b, sem_b)

            def ebody(gp, carry2):
                g = gp * 2
                wait_a()
                pltpu.sync_copy(rows_a, aggsh.at[didx.at[g]], add=True)
                pltpu.async_copy(hh_hbm.at[sidx.at[g + 2]], rows_a, sem_a)
                wait_b()
                pltpu.sync_copy(rows_b, aggsh.at[didx.at[g + 1]], add=True)
                pltpu.async_copy(hh_hbm.at[sidx.at[g + 3]], rows_b, sem_b)
                return carry2

            lax.fori_loop(0, BLK // 2 - 1, ebody, 0)
            wait_a()
            pltpu.sync_copy(rows_a, aggsh.at[didx.at[BLK - 2]], add=True)
            wait_b()
            pltpu.sync_copy(rows_b, aggsh.at[didx.at[BLK - 1]], add=True)
            return carry

        lax.fori_loop(0, NBLK, blk_body, 0)
        plsc.subcore_barrier()
        pltpu.sync_copy(
            aggsh.at[pl.ds(row0, ROWS_PER_TILE)],
            out_hbm.at[pl.ds(c * NPAD + row0, ROWS_PER_TILE)],
        )

    return k(hh, src2d, dst2d)


def _mlp_body(hh_ref, agg_ref, w1_ref, g1_ref, b1_ref, w2_ref, go_ref, bo_ref,
              out_ref, pooled_ref, *p0_ref):
    agg = agg_ref[0:N, :] + agg_ref[NPAD:NPAD + N, :]
    z = hh_ref[...] + agg
    z = jnp.dot(z, w1_ref[...], preferred_element_type=jnp.float32,
                precision=lax.Precision.HIGHEST)
    mu = jnp.mean(z, axis=0, keepdims=True)
    var = jnp.mean((z - mu) ** 2, axis=0, keepdims=True)
    z = g1_ref[...] * (z - mu) * lax.rsqrt(var + 1e-5) + b1_ref[...]
    z = jnp.maximum(z, 0.0)
    z = jnp.dot(z, w2_ref[...], preferred_element_type=jnp.float32,
                precision=lax.Precision.HIGHEST)
    mu = jnp.mean(z, axis=0, keepdims=True)
    var = jnp.mean((z - mu) ** 2, axis=0, keepdims=True)
    z = go_ref[...] * (z - mu) * lax.rsqrt(var + 1e-5) + bo_ref[...]
    z = jnp.maximum(z, 0.0)
    out_ref[...] = z
    pooled_ref[...] = jnp.sum(z, axis=0, keepdims=True)
    if p0_ref:
        p0_ref[0][...] = jnp.sum(hh_ref[...], axis=0, keepdims=True)


def _mlp_tc(hh, aggflat, w1, g1, b1, w2, go, bo, first):
    out_shape = [
        jax.ShapeDtypeStruct((N, D), jnp.float32),
        jax.ShapeDtypeStruct((1, D), jnp.float32),
    ]
    if first:
        out_shape.append(jax.ShapeDtypeStruct((1, D), jnp.float32))
    return pl.pallas_call(
        _mlp_body,
        out_shape=out_shape,
    )(hh, aggflat, w1, g1.reshape(1, D), b1.reshape(1, D),
      w2, go.reshape(1, D), bo.reshape(1, D))


def _readout_body(p_ref, wp_ref, bp_ref, out_ref):
    p = p_ref[...]
    wp = wp_ref[...]
    acc = jnp.sum(bp_ref[...], axis=0, keepdims=True)
    for i in range(NUM_LAYERS + 1):
        acc = acc + jnp.dot(p[i:i + 1, :], wp[i], preferred_element_type=jnp.float32,
                            precision=lax.Precision.HIGHEST)
    out_ref[...] = acc


def _readout_tc(pooled_stack, wp_stack, bp_stack):
    return pl.pallas_call(
        _readout_body,
        out_shape=jax.ShapeDtypeStruct((1, D), jnp.float32),
    )(pooled_stack, wp_stack, bp_stack)


def kernel(h, edge_index, params):
    src = edge_index[0]
    dst = edge_index[1]
    npad_e = EPAD - E
    pad_iota = jnp.arange(npad_e, dtype=jnp.int32)
    src_p = jnp.concatenate([src, pad_iota % N])
    dst_p = jnp.concatenate([dst, N + pad_iota % (NPAD - N)])
    src2d = src_p.reshape(NW * NCHUNKS, CHUNK)
    dst2d = dst_p.reshape(NW * NCHUNKS, CHUNK)
    hh = h
    pooled = []
    for i in range(NUM_LAYERS):
        aggflat = _aggregate_sc(hh, src2d, dst2d)
        outs = _mlp_tc(hh, aggflat, params[f"W1_{i}"], params[f"g1_{i}"],
                       params[f"b1_{i}"], params[f"W2_{i}"], params[f"go_{i}"],
                       params[f"bo_{i}"], first=(i == 0))
        if i == 0:
            hh, p, p0 = outs
            pooled.append(p0)
        else:
            hh, p = outs
        pooled.append(p)
    pooled_stack = jnp.concatenate(pooled, axis=0)
    wp_stack = jnp.stack([params[f"Wp_{i}"] for i in range(NUM_LAYERS + 1)])
    bp_stack = jnp.stack([params[f"bp_{i}"] for i in range(NUM_LAYERS + 1)])
    return _readout_tc(pooled_stack, wp_stack, bp_stack)
